# baseline (speedup 1.0000x reference)
import jax
import jax.numpy as jnp
from jax import lax
from jax.experimental import pallas as pl
from jax.experimental.pallas import tpu as pltpu
from jax.experimental.pallas import tpu_sc as plsc

_NUM_CORES = 2
_NUM_SUBCORES = 16
_NW = _NUM_CORES * _NUM_SUBCORES
_L = 16

_N0 = 16384
_N1 = 200
_D = 10
_NZ = 54
_ZPAD = 64
_NTB = _N0 // 128
_TB_PER_W = _NTB // _NW
_NTS = _N1 // 8
_SB = 5
_NSUB = _NTS // _SB
_NJ = _TB_PER_W * _NSUB


def _sc_body(w_hbm, idx_hbm, out_hbm, w_v, wt_v, idx2_v, stg_v, sem_in, sem_out):
    wid = lax.axis_index("s") * _NUM_CORES + lax.axis_index("c")

    pltpu.sync_copy(w_hbm, w_v)

    iota = lax.iota(jnp.int32, _L)

    @plsc.parallel_loop(0, _NZ, unroll=1)
    def build_wt(z):
        for k in range(_D):
            addr = jnp.broadcast_to(z * _D + k, (_L,))
            vec = plsc.load_gather(w_v, [addr])
            wt_v[pl.ds((k * _ZPAD + z) * _L, _L)] = vec

    def idx_copy(j, ib):
        tbl, sb = divmod(j, _NSUB)
        tb = wid * _TB_PER_W + tbl
        return pltpu.make_async_copy(
            idx_hbm.at[pl.ds(sb * _SB, _SB), tb, :], idx2_v.at[ib], sem_in
        )

    def out_copy(j, k, b):
        tbl, sb = divmod(j, _NSUB)
        tb = wid * _TB_PER_W + tbl
        return pltpu.make_async_copy(
            stg_v.at[b, k],
            out_hbm.at[k, pl.ds(sb * _SB, _SB), tb, :],
            sem_out,
        )

    idx_copy(0, 0).start()

    for j in range(_NJ):
        ib = j % 2
        if j + 1 < _NJ:
            idx_copy(j + 1, 1 - ib).start()
        idx_copy(j, ib).wait()
        if j >= 2:
            for k in range(_D):
                out_copy(j - 2, k, ib).wait()

        @plsc.parallel_loop(0, _SB * 64, unroll=2)
        def sub_body(i):
            ts = i >> 6
            col = jnp.bitwise_and(i, 63) * _L
            z = idx2_v[ib, ts, pl.ds(col, _L)]
            zi = z * _L + iota
            for k in range(_D):
                v = plsc.load_gather(wt_v, [zi + k * (_ZPAD * _L)])
                stg_v[ib, k, ts, pl.ds(col, _L)] = v

        for k in range(_D):
            out_copy(j, k, ib).start()

    for j in (_NJ - 2, _NJ - 1):
        for k in range(_D):
            out_copy(j, k, j % 2).wait()


@jax.jit
def _lookup(idx_phys, w_flat):
    mesh = plsc.VectorSubcoreMesh(core_axis_name="c", subcore_axis_name="s")
    f = pl.kernel(
        _sc_body,
        out_type=jax.ShapeDtypeStruct((_D, _NTS, _NTB, 1024), jnp.float32),
        mesh=mesh,
        scratch_types=[
            pltpu.VMEM((_NZ * _D,), jnp.float32),
            pltpu.VMEM((_D * _ZPAD * _L,), jnp.float32),
            pltpu.VMEM((2, _SB, 1024), jnp.int32),
            pltpu.VMEM((2, _D, _SB, 1024), jnp.float32),
            pltpu.SemaphoreType.DMA,
            pltpu.SemaphoreType.DMA,
        ],
        compiler_params=pltpu.CompilerParams(
            needs_layout_passes=False, use_tc_tiling_on_sc=False
        ),
    )
    return f(w_flat, idx_phys)


def kernel(atomic_number, W):
    idx_phys = (
        atomic_number.astype(jnp.int32)
        .reshape(_NTB, 128, _NTS, 8)
        .transpose(2, 0, 3, 1)
        .reshape(_NTS, _NTB, 1024)
    )
    out_phys = _lookup(idx_phys, W.reshape(-1))
    out5 = out_phys.reshape(_D, _NTS, _NTB, 8, 128)
    return out5.transpose(2, 4, 1, 3, 0).reshape(_N0, _N1, _D)
